# Initial kernel scaffold; baseline (speedup 1.0000x reference)
import jax
import jax.numpy as jnp
from jax.experimental import pallas as pl


def kernel(x, pos_emb):
    raise NotImplementedError("write your pallas kernel here")


# speedup vs baseline: 3.3913x; 3.3913x over previous
import jax
import jax.numpy as jnp
from jax.experimental import pallas as pl


def _copy_block(emb_ref, out_ref):
    out_ref[0, :, :] = emb_ref[:, :]


def kernel(x, pos_emb):
    T = x.shape[1]
    D = pos_emb.shape[1]
    R = 512
    out = pl.pallas_call(
        _copy_block,
        grid=(T // R,),
        in_specs=[pl.BlockSpec((R, D), lambda i: (i, 0))],
        out_specs=pl.BlockSpec((1, R, D), lambda i: (0, i, 0)),
        out_shape=jax.ShapeDtypeStruct((1, T, D), pos_emb.dtype),
    )(pos_emb)
    return out


# speedup vs baseline: 3.7174x; 1.0962x over previous
import jax
import jax.numpy as jnp
from jax.experimental import pallas as pl


def _copy_block(emb_ref, out_ref):
    out_ref[0, :, :] = emb_ref[:, :]


def kernel(x, pos_emb):
    T = x.shape[1]
    D = pos_emb.shape[1]
    R = 1024
    out = pl.pallas_call(
        _copy_block,
        grid=(T // R,),
        in_specs=[pl.BlockSpec((R, D), lambda i: (i, 0))],
        out_specs=pl.BlockSpec((1, R, D), lambda i: (0, i, 0)),
        out_shape=jax.ShapeDtypeStruct((1, T, D), pos_emb.dtype),
    )(pos_emb)
    return out


# speedup vs baseline: 4.1231x; 1.1091x over previous
import jax
import jax.numpy as jnp
from jax.experimental import pallas as pl


def _copy_block(emb_ref, out_ref):
    out_ref[0, :, :] = emb_ref[:, :]


def kernel(x, pos_emb):
    T = x.shape[1]
    D = pos_emb.shape[1]
    R = 2048
    out = pl.pallas_call(
        _copy_block,
        grid=(T // R,),
        in_specs=[pl.BlockSpec((R, D), lambda i: (i, 0))],
        out_specs=pl.BlockSpec((1, R, D), lambda i: (0, i, 0)),
        out_shape=jax.ShapeDtypeStruct((1, T, D), pos_emb.dtype),
    )(pos_emb)
    return out
